# baseline (speedup 1.0000x reference)
import functools

import jax
import jax.numpy as jnp
from jax import lax
from jax.experimental import pallas as pl
from jax.experimental.pallas import tpu as pltpu
from jax.experimental.pallas import tpu_sc as plsc

B = 4096
D = 64
NCAT = 26
NNUM = 13
CATV = 1000
L = 200
NPRE = NCAT + NNUM
NOUT = NPRE + 2 * L
NPAD = 440
DPAD = 2 * D

NC = 2
NS = 16
NW = NC * NS

_MESH = plsc.VectorSubcoreMesh(core_axis_name="c", subcore_axis_name="s")
_SC_PARAMS = pltpu.CompilerParams(use_tc_tiling_on_sc=False)

_CAT_ROWS = B * NCAT
_CAT_PER_W = _CAT_ROWS // NW
_CAT_CHUNK = 208
_NCHUNK = _CAT_PER_W // _CAT_CHUNK
_B_PER_W = B // NW
_IDX_PER_W = _B_PER_W * L


@functools.partial(
    pl.kernel,
    out_type=[
        jax.ShapeDtypeStruct((_CAT_ROWS, D), jnp.float32),
        jax.ShapeDtypeStruct((B, NPAD, DPAD), jnp.float32),
    ],
    mesh=_MESH,
    scratch_types=[
        pltpu.VMEM((L, DPAD), jnp.float32),
        pltpu.VMEM((L, DPAD), jnp.float32),
        pltpu.VMEM((_CAT_CHUNK, D), jnp.float32),
        pltpu.VMEM((_CAT_CHUNK, D), jnp.float32),
        pltpu.VMEM((_IDX_PER_W,), jnp.int32),
        pltpu.VMEM((_IDX_PER_W,), jnp.int32),
        pltpu.SemaphoreType.DMA,
        pltpu.SemaphoreType.DMA,
        pltpu.SemaphoreType.DMA,
        pltpu.SemaphoreType.DMA,
    ],
    compiler_params=_SC_PARAMS,
)
def _gather_all(cat_idx_hbm, cat_tab_hbm, aidx_hbm, sidx_hbm, atab_hbm,
                stab_hbm, cat_out_hbm, out_hbm, a0_v, a1_v, s0_v, s1_v,
                aidx_v, sidx_v, sem_a0, sem_a1, sem_s0, sem_s1):
    wid = lax.axis_index("s") * NC + lax.axis_index("c")
    a_slot = (a0_v, a1_v)
    s_slot = (s0_v, s1_v)
    ga_sem = (sem_a0, sem_a1)
    gs_sem = (sem_s0, sem_s1)

    pltpu.sync_copy(cat_idx_hbm.at[pl.ds(wid * _CAT_PER_W, _CAT_PER_W)],
                    sidx_v.at[pl.ds(0, _CAT_PER_W)])

    def cat_issue(c, k):
        idx = sidx_v.at[pl.ds(c * _CAT_CHUNK, _CAT_CHUNK)]
        pltpu.async_copy(cat_tab_hbm.at[idx], s_slot[k], gs_sem[k])

    def cat_wait(k):
        pltpu.make_async_copy(
            cat_tab_hbm.at[pl.ds(0, _CAT_CHUNK)], s_slot[k],
            gs_sem[k]).wait()

    cat_issue(0, 0)
    for c in range(_NCHUNK):
        if c + 1 < _NCHUNK:
            cat_issue(c + 1, (c + 1) % 2)
        cat_wait(c % 2)
        pltpu.sync_copy(
            s_slot[c % 2],
            cat_out_hbm.at[pl.ds(wid * _CAT_PER_W + c * _CAT_CHUNK,
                                 _CAT_CHUNK)])

    pltpu.sync_copy(aidx_hbm.at[pl.ds(wid * _IDX_PER_W, _IDX_PER_W)], aidx_v)
    pltpu.sync_copy(sidx_hbm.at[pl.ds(wid * _IDX_PER_W, _IDX_PER_W)], sidx_v)

    def issue(k, i):
        ia = aidx_v.at[pl.ds(i * L, L)]
        js = sidx_v.at[pl.ds(i * L, L)]
        pltpu.async_copy(atab_hbm.at[ia], a_slot[k], ga_sem[k])
        pltpu.async_copy(stab_hbm.at[js], s_slot[k].at[pl.ds(0, L)],
                         gs_sem[k])

    def wait(k):
        pltpu.make_async_copy(atab_hbm.at[pl.ds(0, L)], a_slot[k],
                              ga_sem[k]).wait()
        pltpu.make_async_copy(stab_hbm.at[pl.ds(0, L)],
                              s_slot[k].at[pl.ds(0, L)], gs_sem[k]).wait()

    def store(k, i):
        b = wid * _B_PER_W + i
        pltpu.sync_copy(a_slot[k], out_hbm.at[b, pl.ds(NPRE, L)])
        pltpu.sync_copy(s_slot[k].at[pl.ds(0, L)],
                        out_hbm.at[b, pl.ds(NPRE + L, L), pl.ds(0, D)])

    issue(0, 0)

    def body(j, carry):
        i0 = 2 * j
        issue(1, i0 + 1)
        wait(0)
        store(0, i0)
        issue(0, i0 + 2)
        wait(1)
        store(1, i0 + 1)
        return carry

    lax.fori_loop(0, _B_PER_W // 2 - 1, body, 0)
    issue(1, _B_PER_W - 1)
    wait(0)
    store(0, _B_PER_W - 2)
    wait(1)
    store(1, _B_PER_W - 1)


_TB = 256


def _ln(h, g, b, axis):
    mu = jnp.mean(h, axis=axis, keepdims=True)
    d = h - mu
    var = jnp.mean(d * d, axis=axis, keepdims=True)
    return d * lax.rsqrt(var + 1e-5) * g + b


def _gelu(x):
    return 0.5 * x * (1.0 + lax.erf(x * 0.7071067811865476))


def _tc_feat(out_in_ref, cat_ref, nf_ref, numw_ref, numb_ref, w1_ref,
             b1_ref, g_ref, be_ref, out_ref):
    w1 = w1_ref[...]
    b1 = b1_ref[...]
    g = g_ref[...]
    be = be_ref[...]
    x = cat_ref[...]
    h = lax.dot_general(x, w1, (((1,), (1,)), ((), ())),
                        preferred_element_type=jnp.float32) + b1
    fc = _ln(_gelu(h), g, be, 1).reshape(_TB, NCAT, D)
    nf = nf_ref[...]
    A = lax.dot_general(numw_ref[...], w1, (((1,), (1,)), ((), ())),
                        preferred_element_type=jnp.float32)
    c = lax.dot_general(numb_ref[...], w1, (((1,), (1,)), ((), ())),
                        preferred_element_type=jnp.float32) + b1
    hn = nf[:, :, None] * A[None] + c[None]
    fn = _ln(_gelu(hn), g[None], be[None], 2)
    feat = jnp.concatenate([fc, fn], axis=1)
    feat = jnp.concatenate(
        [feat, jnp.zeros((_TB, NPRE, D), jnp.float32)], axis=2)
    out_ref[...] = jnp.concatenate(
        [feat, out_in_ref[:, NPRE:NPRE + 1, :]], axis=1)


def _feat(out_buf, cat_rows, num_feats, num_W, num_b, W1, b1, gamma, beta):
    nblk = B // _TB
    return pl.pallas_call(
        _tc_feat,
        grid=(nblk,),
        in_specs=[
            pl.BlockSpec((_TB, NPRE + 1, DPAD), lambda i: (i, 0, 0)),
            pl.BlockSpec((_TB * NCAT, D), lambda i: (i, 0)),
            pl.BlockSpec((_TB, NNUM), lambda i: (i, 0)),
            pl.BlockSpec((NNUM, D), lambda i: (0, 0)),
            pl.BlockSpec((NNUM, D), lambda i: (0, 0)),
            pl.BlockSpec((D, D), lambda i: (0, 0)),
            pl.BlockSpec((1, D), lambda i: (0, 0)),
            pl.BlockSpec((1, D), lambda i: (0, 0)),
            pl.BlockSpec((1, D), lambda i: (0, 0)),
        ],
        out_specs=pl.BlockSpec((_TB, NPRE + 1, DPAD), lambda i: (i, 0, 0)),
        out_shape=jax.ShapeDtypeStruct((B, NPAD, DPAD), jnp.float32),
        input_output_aliases={0: 0},
    )(out_buf, cat_rows, num_feats, num_W, num_b, W1, b1, gamma, beta)


def kernel(cat_feats, num_feats, atom_history, sem_history, cat_tables,
           num_W, num_b, W1, b1, gamma, beta, atom_table, sem_table):
    cat_idx = (cat_feats.astype(jnp.int32)
               + jnp.arange(NCAT, dtype=jnp.int32)[None, :] * CATV).reshape(-1)
    atom_idx = atom_history.astype(jnp.int32).reshape(-1)
    sem_idx = sem_history.astype(jnp.int32).reshape(-1)
    cat_flat = cat_tables.reshape(NCAT * CATV, D)
    atom_pad = jnp.pad(atom_table, ((0, 7), (0, D)))

    cat_rows, out_buf = _gather_all(cat_idx, cat_flat, atom_idx, sem_idx,
                                    atom_pad, sem_table)
    out_buf = _feat(out_buf, cat_rows, num_feats, num_W, num_b, W1,
                    b1.reshape(1, D), gamma.reshape(1, D), beta.reshape(1, D))
    return out_buf[:, :NOUT, :D]


# speedup vs baseline: 1.0555x; 1.0555x over previous
import functools

import jax
import jax.numpy as jnp
from jax import lax
from jax.experimental import pallas as pl
from jax.experimental.pallas import tpu as pltpu
from jax.experimental.pallas import tpu_sc as plsc

B = 4096
D = 64
NCAT = 26
NNUM = 13
CATV = 1000
L = 200
NPRE = NCAT + NNUM
NOUT = NPRE + 2 * L
NPAD = 440
DPAD = 2 * D

NC = 2
NS = 16
NW = NC * NS

_MESH = plsc.VectorSubcoreMesh(core_axis_name="c", subcore_axis_name="s")
_SC_PARAMS = pltpu.CompilerParams(use_tc_tiling_on_sc=False)

_CAT_ROWS = B * NCAT
_CAT_PER_W = _CAT_ROWS // NW
_CAT_CHUNK = 208
_NCHUNK = _CAT_PER_W // _CAT_CHUNK
_B_PER_W = B // NW
_IDX_PER_W = _B_PER_W * L


@functools.partial(
    pl.kernel,
    out_type=[
        jax.ShapeDtypeStruct((_CAT_ROWS, D), jnp.float32),
        jax.ShapeDtypeStruct((B, NPAD, DPAD), jnp.float32),
    ],
    mesh=_MESH,
    scratch_types=[
        pltpu.VMEM((_CAT_CHUNK, D), jnp.float32),
        pltpu.VMEM((_CAT_CHUNK, D), jnp.float32),
        pltpu.VMEM((L, D), jnp.float32),
        pltpu.VMEM((L, D), jnp.float32),
        pltpu.VMEM((_IDX_PER_W,), jnp.int32),
        pltpu.VMEM((_IDX_PER_W,), jnp.int32),
        pltpu.SemaphoreType.DMA,
        pltpu.SemaphoreType.DMA,
        pltpu.SemaphoreType.DMA,
        pltpu.SemaphoreType.DMA,
    ],
    compiler_params=_SC_PARAMS,
)
def _gather_all(cat_idx_hbm, cat_tab_hbm, aidx_hbm, sidx_hbm, atab_hbm,
                stab_hbm, cat_out_hbm, out_hbm, a0_v, a1_v, s0_v, s1_v,
                aidx_v, sidx_v, sem_a0, sem_a1, sem_s0, sem_s1):
    wid = lax.axis_index("s") * NC + lax.axis_index("c")
    a_slot = (a0_v, a1_v)
    s_slot = (s0_v, s1_v)
    ga_sem = (sem_a0, sem_a1)
    gs_sem = (sem_s0, sem_s1)

    pltpu.sync_copy(cat_idx_hbm.at[pl.ds(wid * _CAT_PER_W, _CAT_PER_W)],
                    sidx_v.at[pl.ds(0, _CAT_PER_W)])

    def cat_issue(c, k):
        idx = sidx_v.at[pl.ds(c * _CAT_CHUNK, _CAT_CHUNK)]
        pltpu.async_copy(cat_tab_hbm.at[idx], a_slot[k], ga_sem[k])

    def cat_wait(k):
        pltpu.make_async_copy(
            cat_tab_hbm.at[pl.ds(0, _CAT_CHUNK)], a_slot[k],
            ga_sem[k]).wait()

    cat_issue(0, 0)
    for c in range(_NCHUNK):
        if c + 1 < _NCHUNK:
            cat_issue(c + 1, (c + 1) % 2)
        cat_wait(c % 2)
        pltpu.sync_copy(
            a_slot[c % 2],
            cat_out_hbm.at[pl.ds(wid * _CAT_PER_W + c * _CAT_CHUNK,
                                 _CAT_CHUNK)])

    pltpu.sync_copy(aidx_hbm.at[pl.ds(wid * _IDX_PER_W, _IDX_PER_W)], aidx_v)
    pltpu.sync_copy(sidx_hbm.at[pl.ds(wid * _IDX_PER_W, _IDX_PER_W)], sidx_v)

    def issue(k, i):
        ia = aidx_v.at[pl.ds(i * L, L)]
        js = sidx_v.at[pl.ds(i * L, L)]
        pltpu.async_copy(atab_hbm.at[ia], a_slot[k].at[pl.ds(0, L)],
                         ga_sem[k])
        pltpu.async_copy(stab_hbm.at[js], s_slot[k], gs_sem[k])

    def wait(k):
        pltpu.make_async_copy(atab_hbm.at[pl.ds(0, L)],
                              a_slot[k].at[pl.ds(0, L)], ga_sem[k]).wait()
        pltpu.make_async_copy(stab_hbm.at[pl.ds(0, L)], s_slot[k],
                              gs_sem[k]).wait()

    def store(k, i):
        b = wid * _B_PER_W + i
        pltpu.sync_copy(a_slot[k].at[pl.ds(0, L)],
                        out_hbm.at[b, pl.ds(NPRE, L), pl.ds(0, D)])
        pltpu.sync_copy(s_slot[k],
                        out_hbm.at[b, pl.ds(NPRE + L, L), pl.ds(0, D)])

    issue(0, 0)

    def body(j, carry):
        i0 = 2 * j
        issue(1, i0 + 1)
        wait(0)
        store(0, i0)
        issue(0, i0 + 2)
        wait(1)
        store(1, i0 + 1)
        return carry

    lax.fori_loop(0, _B_PER_W // 2 - 1, body, 0)
    issue(1, _B_PER_W - 1)
    wait(0)
    store(0, _B_PER_W - 2)
    wait(1)
    store(1, _B_PER_W - 1)


_TB = 256


def _ln(h, g, b, axis):
    mu = jnp.mean(h, axis=axis, keepdims=True)
    d = h - mu
    var = jnp.mean(d * d, axis=axis, keepdims=True)
    return d * lax.rsqrt(var + 1e-5) * g + b


def _gelu(x):
    return 0.5 * x * (1.0 + lax.erf(x * 0.7071067811865476))


def _tc_feat(out_in_ref, cat_ref, nf_ref, numw_ref, numb_ref, w1_ref,
             b1_ref, g_ref, be_ref, out_ref):
    w1 = w1_ref[...]
    b1 = b1_ref[...]
    g = g_ref[...]
    be = be_ref[...]
    x = cat_ref[...]
    h = lax.dot_general(x, w1, (((1,), (1,)), ((), ())),
                        preferred_element_type=jnp.float32) + b1
    fc = _ln(_gelu(h), g, be, 1).reshape(_TB, NCAT, D)
    nf = nf_ref[...]
    A = lax.dot_general(numw_ref[...], w1, (((1,), (1,)), ((), ())),
                        preferred_element_type=jnp.float32)
    c = lax.dot_general(numb_ref[...], w1, (((1,), (1,)), ((), ())),
                        preferred_element_type=jnp.float32) + b1
    hn = nf[:, :, None] * A[None] + c[None]
    fn = _ln(_gelu(hn), g[None], be[None], 2)
    feat = jnp.concatenate([fc, fn], axis=1)
    feat = jnp.concatenate(
        [feat, jnp.zeros((_TB, NPRE, D), jnp.float32)], axis=2)
    out_ref[...] = jnp.concatenate(
        [feat, out_in_ref[:, NPRE:NPRE + 1, :]], axis=1)


def _feat(out_buf, cat_rows, num_feats, num_W, num_b, W1, b1, gamma, beta):
    nblk = B // _TB
    return pl.pallas_call(
        _tc_feat,
        grid=(nblk,),
        in_specs=[
            pl.BlockSpec((_TB, NPRE + 1, DPAD), lambda i: (i, 0, 0)),
            pl.BlockSpec((_TB * NCAT, D), lambda i: (i, 0)),
            pl.BlockSpec((_TB, NNUM), lambda i: (i, 0)),
            pl.BlockSpec((NNUM, D), lambda i: (0, 0)),
            pl.BlockSpec((NNUM, D), lambda i: (0, 0)),
            pl.BlockSpec((D, D), lambda i: (0, 0)),
            pl.BlockSpec((1, D), lambda i: (0, 0)),
            pl.BlockSpec((1, D), lambda i: (0, 0)),
            pl.BlockSpec((1, D), lambda i: (0, 0)),
        ],
        out_specs=pl.BlockSpec((_TB, NPRE + 1, DPAD), lambda i: (i, 0, 0)),
        out_shape=jax.ShapeDtypeStruct((B, NPAD, DPAD), jnp.float32),
        input_output_aliases={0: 0},
    )(out_buf, cat_rows, num_feats, num_W, num_b, W1, b1, gamma, beta)


def kernel(cat_feats, num_feats, atom_history, sem_history, cat_tables,
           num_W, num_b, W1, b1, gamma, beta, atom_table, sem_table):
    cat_idx = (cat_feats.astype(jnp.int32)
               + jnp.arange(NCAT, dtype=jnp.int32)[None, :] * CATV).reshape(-1)
    atom_idx = atom_history.astype(jnp.int32).reshape(-1)
    sem_idx = sem_history.astype(jnp.int32).reshape(-1)
    cat_flat = cat_tables.reshape(NCAT * CATV, D)

    cat_rows, out_buf = _gather_all(cat_idx, cat_flat, atom_idx, sem_idx,
                                    atom_table, sem_table)
    out_buf = _feat(out_buf, cat_rows, num_feats, num_W, num_b, W1,
                    b1.reshape(1, D), gamma.reshape(1, D), beta.reshape(1, D))
    return out_buf[:, :NOUT, :D]
